# baseline (speedup 1.0000x reference)
import jax
import jax.numpy as jnp
from jax.experimental import pallas as pl
from jax.experimental.pallas import tpu as pltpu

_BLOCK = 4096


def _gcn_kernel(x_ref, adj_ref, w_ref, out_ref):
    c = jax.lax.dot_general(
        adj_ref[...],
        w_ref[...],
        (((1,), (1,)), ((), ())),
        preferred_element_type=jnp.float32,
        precision=jax.lax.Precision.HIGHEST,
    )
    out_ref[...] = jax.lax.dot_general(
        c,
        x_ref[...],
        (((1,), (1,)), ((), ())),
        preferred_element_type=jnp.float32,
    )


def kernel(input, adj, weight):
    x = input.reshape(-1, weight.shape[0])
    n = x.shape[0]
    m = adj.shape[0]
    out = pl.pallas_call(
        _gcn_kernel,
        grid=(pl.cdiv(n, _BLOCK),),
        in_specs=[
            pl.BlockSpec((_BLOCK, x.shape[1]), lambda i: (i, 0)),
            pl.BlockSpec(adj.shape, lambda i: (0, 0)),
            pl.BlockSpec(weight.shape, lambda i: (0, 0)),
        ],
        out_specs=pl.BlockSpec((m, _BLOCK), lambda i: (0, i)),
        out_shape=jax.ShapeDtypeStruct((m, n), jnp.float32),
        compiler_params=pltpu.CompilerParams(
            dimension_semantics=("parallel",),
        ),
    )(x, adj, weight)
    return (out, weight)


# speedup vs baseline: 2.0058x; 2.0058x over previous
import jax
import jax.numpy as jnp
from jax.experimental import pallas as pl
from jax.experimental.pallas import tpu as pltpu

_BLOCK = 4096


def _gcn_kernel(x_ref, adj_ref, w_ref, y_ref):
    ct = jax.lax.dot_general(
        w_ref[...],
        adj_ref[...],
        (((1,), (1,)), ((), ())),
        preferred_element_type=jnp.float32,
        precision=jax.lax.Precision.HIGHEST,
    )
    y_ref[...] = jax.lax.dot_general(
        x_ref[...],
        ct,
        (((1,), (0,)), ((), ())),
        preferred_element_type=jnp.float32,
    )


def kernel(input, adj, weight):
    x = input.reshape(-1, weight.shape[0])
    n = x.shape[0]
    m = adj.shape[0]
    y = pl.pallas_call(
        _gcn_kernel,
        grid=(pl.cdiv(n, _BLOCK),),
        in_specs=[
            pl.BlockSpec((_BLOCK, x.shape[1]), lambda i: (i, 0)),
            pl.BlockSpec(adj.shape, lambda i: (0, 0)),
            pl.BlockSpec(weight.shape, lambda i: (0, 0)),
        ],
        out_specs=pl.BlockSpec((_BLOCK, m), lambda i: (i, 0)),
        out_shape=jax.ShapeDtypeStruct((n, m), jnp.float32),
        compiler_params=pltpu.CompilerParams(
            dimension_semantics=("parallel",),
        ),
    )(x, adj, weight)
    return (y.T, weight)


# speedup vs baseline: 2.4657x; 1.2293x over previous
import jax
import jax.numpy as jnp
from jax.experimental import pallas as pl
from jax.experimental.pallas import tpu as pltpu

_BLOCK = 8192


def _gcn_kernel(x_ref, adj_ref, w_ref, y_ref):
    ct = jax.lax.dot_general(
        w_ref[...],
        adj_ref[...],
        (((1,), (1,)), ((), ())),
        preferred_element_type=jnp.float32,
        precision=jax.lax.Precision.HIGHEST,
    )
    y_ref[...] = jax.lax.dot_general(
        x_ref[...],
        ct,
        (((1,), (0,)), ((), ())),
        preferred_element_type=jnp.float32,
    )


def kernel(input, adj, weight):
    x = input.reshape(-1, weight.shape[0])
    n = x.shape[0]
    m = adj.shape[0]
    y = pl.pallas_call(
        _gcn_kernel,
        grid=(pl.cdiv(n, _BLOCK),),
        in_specs=[
            pl.BlockSpec((_BLOCK, x.shape[1]), lambda i: (i, 0)),
            pl.BlockSpec(adj.shape, lambda i: (0, 0)),
            pl.BlockSpec(weight.shape, lambda i: (0, 0)),
        ],
        out_specs=pl.BlockSpec((_BLOCK, m), lambda i: (i, 0)),
        out_shape=jax.ShapeDtypeStruct((n, m), jnp.float32),
        compiler_params=pltpu.CompilerParams(
            dimension_semantics=("parallel",),
        ),
    )(x, adj, weight)
    return (y.T, weight)


# speedup vs baseline: 2.5840x; 1.0480x over previous
import jax
import jax.numpy as jnp
from jax.experimental import pallas as pl
from jax.experimental.pallas import tpu as pltpu

_BLOCK = 16384


def _gcn_kernel(x_ref, adj_ref, w_ref, y_ref):
    ct = jax.lax.dot_general(
        w_ref[...],
        adj_ref[...],
        (((1,), (1,)), ((), ())),
        preferred_element_type=jnp.float32,
        precision=jax.lax.Precision.HIGHEST,
    )
    y_ref[...] = jax.lax.dot_general(
        x_ref[...],
        ct,
        (((1,), (0,)), ((), ())),
        preferred_element_type=jnp.float32,
    )


def kernel(input, adj, weight):
    x = input.reshape(-1, weight.shape[0])
    n = x.shape[0]
    m = adj.shape[0]
    y = pl.pallas_call(
        _gcn_kernel,
        grid=(pl.cdiv(n, _BLOCK),),
        in_specs=[
            pl.BlockSpec((_BLOCK, x.shape[1]), lambda i: (i, 0)),
            pl.BlockSpec(adj.shape, lambda i: (0, 0)),
            pl.BlockSpec(weight.shape, lambda i: (0, 0)),
        ],
        out_specs=pl.BlockSpec((_BLOCK, m), lambda i: (i, 0)),
        out_shape=jax.ShapeDtypeStruct((n, m), jnp.float32),
        compiler_params=pltpu.CompilerParams(
            dimension_semantics=("parallel",),
        ),
    )(x, adj, weight)
    return (y.T, weight)
